# Initial kernel scaffold; baseline (speedup 1.0000x reference)
import jax
import jax.numpy as jnp
from jax.experimental import pallas as pl


def kernel(x, w, b):
    raise NotImplementedError("write your pallas kernel here")


# speedup vs baseline: 2.2438x; 2.2438x over previous
import functools

import jax
import jax.numpy as jnp
from jax import lax
from jax.experimental import pallas as pl
from jax.experimental.pallas import tpu as pltpu

_DROPOUT_P = 0.5
_DROPOUT_SEED = 1234
_GOLDEN = 0x9E3779B9


def _fused_kernel(x_ref, w_ref, b_ref, o_ref, *, n_total, threshold, seed_u,
                  scale):
    tm, tn = o_ref.shape
    acc = lax.dot_general(
        x_ref[...], w_ref[...],
        dimension_numbers=(((1,), (1,)), ((), ())),
        preferred_element_type=jnp.float32)
    y = jnp.maximum(acc + b_ref[...], 0.0)

    row_off = (pl.program_id(0) * tm).astype(jnp.uint32)
    col_off = (pl.program_id(1) * tn).astype(jnp.uint32)
    rows = lax.broadcasted_iota(jnp.int32, (tm, tn), 0).astype(jnp.uint32) + row_off
    cols = lax.broadcasted_iota(jnp.int32, (tm, tn), 1).astype(jnp.uint32) + col_off
    idx = rows * jnp.uint32(n_total) + cols
    h = idx ^ jnp.uint32(seed_u)
    h = h ^ (h >> 16)
    h = h * jnp.uint32(0x85EBCA6B)
    h = h ^ (h >> 13)
    h = h * jnp.uint32(0xC2B2AE35)
    h = h ^ (h >> 16)
    keep = (h & jnp.uint32(0x00FFFFFF)) >= jnp.uint32(threshold)
    o_ref[...] = jnp.where(keep, y * jnp.float32(scale), 0.0)


def kernel(x, w, b):
    B, K = x.shape
    N, Kw = w.shape
    assert Kw == K

    bm = min(1024, B)
    bn = min(1024, N)
    grid = (B // bm, N // bn)

    xb = x.astype(jnp.bfloat16)
    wb = w.astype(jnp.bfloat16)
    b2 = b.reshape(1, N).astype(jnp.float32)

    seed_u = (_DROPOUT_SEED * _GOLDEN) & 0xFFFFFFFF
    threshold = int(_DROPOUT_P * (1 << 24))
    body = functools.partial(
        _fused_kernel, n_total=N, threshold=threshold, seed_u=seed_u,
        scale=1.0 / (1.0 - _DROPOUT_P))

    out = pl.pallas_call(
        body,
        grid=grid,
        in_specs=[
            pl.BlockSpec((bm, K), lambda i, j: (i, 0)),
            pl.BlockSpec((bn, K), lambda i, j: (j, 0)),
            pl.BlockSpec((1, bn), lambda i, j: (0, j)),
        ],
        out_specs=pl.BlockSpec((bm, bn), lambda i, j: (i, j)),
        out_shape=jax.ShapeDtypeStruct((B, N), jnp.float32),
        compiler_params=pltpu.CompilerParams(
            dimension_semantics=("parallel", "parallel"),
            vmem_limit_bytes=56 * 1024 * 1024),
    )(xb, wb, b2)
    return out
